# baseline (speedup 1.0000x reference)
import functools

import jax
import jax.numpy as jnp
from jax import lax
from jax.experimental import pallas as pl
from jax.experimental.pallas import tpu as pltpu
from jax.experimental.pallas import tpu_sc as plsc

BEAM = 4
PAD_IDX = 1
EOS_IDX = 2
VOCAB = 1_000_000
BSZ = 16
NROWS = BSZ * BEAM
NC, NS, L = 2, 16, 16
NW = NC * NS
ROWS_PER_W = NROWS // NW
CHUNK = 20_000
NCHUNK = VOCAB // CHUNK
STEPS = CHUNK // L
UNR = 10
NEG = float("-inf")
NEG_I32 = -2147483647

_mesh = plsc.VectorSubcoreMesh(
    core_axis_name="c", subcore_axis_name="s", num_cores=NC, num_subcores=NS
)


def _iota():
    return lax.iota(jnp.int32, L)


def _splat_f(x):
    return jnp.full((L,), x, jnp.float32)


def _extract_f(vec, lane):
    return jnp.max(jnp.where(_iota() == lane, vec, NEG))


def _extract_i(vec, lane):
    return jnp.max(jnp.where(_iota() == lane, vec, NEG_I32))


@functools.partial(
    pl.kernel,
    out_type=[
        jax.ShapeDtypeStruct((NROWS * L,), jnp.float32),
        jax.ShapeDtypeStruct((NROWS * L,), jnp.int32),
    ],
    mesh=_mesh,
    compiler_params=pltpu.CompilerParams(needs_layout_passes=False),
    scratch_types=[
        pltpu.VMEM((CHUNK,), jnp.float32),
        pltpu.VMEM((L,), jnp.float32),
        pltpu.VMEM((L,), jnp.int32),
        pltpu.VMEM((L,), jnp.float32),
        pltpu.VMEM((L,), jnp.int32),
    ],
)
def _stage1(lg_hbm, rv_hbm, ri_hbm, buf, rv, ri, ovf, ovi):
    wid = lax.axis_index("s") * NC + lax.axis_index("c")

    def scan_argmax(_):
        def body(i, carry):
            mv, mi = carry
            base = i * (UNR * L)
            for u in range(UNR):
                v = buf[pl.ds(base + u * L, L)]
                take = v > mv
                step = jnp.full((L,), i * UNR + u, jnp.int32)
                mv = jnp.where(take, v, mv)
                mi = jnp.where(take, step, mi)
            return mv, mi

        return lax.fori_loop(
            0, STEPS // UNR, body, (_splat_f(NEG), jnp.zeros((L,), jnp.int32))
        )

    def drilldown(c):
        def cond(carry):
            k, go = carry
            return go & (k < BEAM)

        def body(carry):
            k, _ = carry
            mv, mi = scan_argmax(None)
            bestv = jnp.max(mv)
            lane = jnp.max(plsc.all_reduce_ffs(mv == bestv))
            step = _extract_i(mi, lane)
            r3 = _extract_f(rv[...], jnp.int32(3))
            ins = bestv > r3

            @pl.when(ins)
            def _():
                vidx = c * CHUNK + step * L + lane
                io = _iota()
                keys = jnp.where(io == 4, bestv, jnp.where(io < 4, rv[...], NEG))
                vals = jnp.where(io == 4, vidx, ri[...])
                sk, sv = plsc.sort_key_val(keys, vals, descending=True)
                rv[...] = sk
                ri[...] = sv
                pos = step * L
                old = buf[pl.ds(pos, L)]
                buf[pl.ds(pos, L)] = jnp.where(io == lane, NEG, old)

            return k + 1, ins

        lax.while_loop(cond, body, (jnp.int32(0), True))

    for rr in range(ROWS_PER_W):
        row = wid * ROWS_PER_W + rr
        rv[...] = _splat_f(NEG)
        ri[...] = jnp.zeros((L,), jnp.int32)

        def chunk_body(c, carry):
            m16, s16 = carry
            pltpu.sync_copy(lg_hbm.at[pl.ds(row * VOCAB + c * CHUNK, CHUNK)], buf)

            @pl.when(c == 0)
            def _():
                v0 = buf[pl.ds(0, L)]
                buf[pl.ds(0, L)] = jnp.where(_iota() == PAD_IDX, NEG, v0)

            def p1(i, acc):
                a0, a1 = acc
                base = i * (UNR * L)
                for u in range(UNR // 2):
                    a0 = jnp.maximum(a0, buf[pl.ds(base + 2 * u * L, L)])
                    a1 = jnp.maximum(a1, buf[pl.ds(base + (2 * u + 1) * L, L)])
                return a0, a1

            a0, a1 = lax.fori_loop(0, STEPS // UNR, p1, (_splat_f(NEG), _splat_f(NEG)))
            cm16 = jnp.maximum(a0, a1)

            def p2(i, acc):
                e0, e1 = acc
                base = i * (UNR * L)
                for u in range(UNR // 2):
                    e0 = e0 + jnp.exp(buf[pl.ds(base + 2 * u * L, L)] - cm16)
                    e1 = e1 + jnp.exp(buf[pl.ds(base + (2 * u + 1) * L, L)] - cm16)
                return e0, e1

            e0, e1 = lax.fori_loop(
                0, STEPS // UNR, p2, (_splat_f(0.0), _splat_f(0.0))
            )
            sc16 = e0 + e1

            m_new = jnp.maximum(m16, cm16)
            s_new = s16 * jnp.exp(m16 - m_new) + sc16 * jnp.exp(cm16 - m_new)

            @pl.when(jnp.max(cm16) > _extract_f(rv[...], jnp.int32(3)))
            def _():
                drilldown(c)

            return m_new, s_new

        m16, s16 = lax.fori_loop(
            0, NCHUNK, chunk_body, (_splat_f(NEG), _splat_f(0.0))
        )

        m_row = jnp.max(m16)
        s_row = jnp.sum(s16 * jnp.exp(m16 - m_row))
        io = _iota()
        ovf[...] = jnp.where(
            io == 4, m_row, jnp.where(io == 5, s_row, rv[...])
        )
        ovi[...] = ri[...]
        pltpu.sync_copy(ovf, rv_hbm.at[pl.ds(row * L, L)])
        pltpu.sync_copy(ovi, ri_hbm.at[pl.ds(row * L, L)])


@functools.partial(
    pl.kernel,
    out_type=[
        jax.ShapeDtypeStruct((BSZ * L,), jnp.int32),
        jax.ShapeDtypeStruct((BSZ * L,), jnp.float32),
        jax.ShapeDtypeStruct((BSZ * L,), jnp.float32),
        jax.ShapeDtypeStruct((BSZ * L,), jnp.int32),
    ],
    mesh=_mesh,
    compiler_params=pltpu.CompilerParams(needs_layout_passes=False),
    scratch_types=[
        pltpu.VMEM((BEAM * L,), jnp.float32),
        pltpu.VMEM((BEAM * L,), jnp.int32),
        pltpu.VMEM((NROWS,), jnp.float32),
        pltpu.VMEM((NROWS,), jnp.float32),
        pltpu.VMEM((NROWS,), jnp.float32),
        pltpu.VMEM((L,), jnp.float32),
        pltpu.VMEM((L,), jnp.float32),
        pltpu.VMEM((L,), jnp.int32),
        pltpu.VMEM((L,), jnp.float32),
        pltpu.VMEM((L,), jnp.int32),
    ],
)
def _stage2(
    rv_hbm, ri_hbm, prev_hbm, osc_hbm, fin_hbm, norm_hbm,
    tok_hbm, sco_hbm, ssc_hbm, ord_hbm,
    vbuf, ibuf, pbuf, obuf, fbuf, nbuf, gf, gi, of, oi,
):
    wid = lax.axis_index("s") * NC + lax.axis_index("c")

    @pl.when(wid < BSZ)
    def _():
        ss = wid
        pltpu.sync_copy(rv_hbm.at[pl.ds(ss * BEAM * L, BEAM * L)], vbuf)
        pltpu.sync_copy(ri_hbm.at[pl.ds(ss * BEAM * L, BEAM * L)], ibuf)
        pltpu.sync_copy(prev_hbm, pbuf)
        pltpu.sync_copy(osc_hbm, obuf)
        pltpu.sync_copy(fin_hbm, fbuf)
        pltpu.sync_copy(norm_hbm, nbuf)

        io = _iota()
        rloc = jnp.right_shift(io, 2)
        slot = jnp.bitwise_and(io, 3)
        rbase = rloc * L
        cv = plsc.load_gather(vbuf, [rbase + slot])
        tok = plsc.load_gather(ibuf, [rbase + slot])
        m = plsc.load_gather(vbuf, [rbase + 4])
        s = plsc.load_gather(vbuf, [rbase + 5])

        bits = plsc.bitcast(s, jnp.int32)
        ef = (jnp.right_shift(bits, 23) - 127).astype(jnp.float32)
        mant = plsc.bitcast(
            jnp.bitwise_or(
                jnp.bitwise_and(bits, jnp.int32(0x007FFFFF)), jnp.int32(0x3F800000)
            ),
            jnp.float32,
        )
        t = mant - 1.0
        x = ef * jnp.float32(0.6931471805599453) + t * (1.0 - t * (0.5 - t / 3.0))
        for _ in range(3):
            x = x + s * jnp.exp(-x) - 1.0
        lse = m + x

        grow = ss * BEAM + rloc
        prev = plsc.load_gather(pbuf, [grow])
        osc = plsc.load_gather(obuf, [grow])
        fin = plsc.load_gather(fbuf, [grow]) > 0.0
        norm = nbuf[...]

        cs = cv - lse + prev
        css = cs / norm
        tok = jnp.where(fin, jnp.int32(EOS_IDX), tok)
        cs = jnp.where(fin, NEG, cs)
        first = slot == 0
        css = jnp.where(fin, jnp.where(first, osc, NEG), css)

        top_vals, top_cands = plsc.sort_key_val(css, io, descending=True)

        gi[...] = tok
        new_tok = plsc.load_gather(gi, [top_cands])
        gf[...] = cs
        new_sco = plsc.load_gather(gf, [top_cands])
        new_ord = jnp.right_shift(top_cands, 2) + ss * BEAM

        oi[...] = new_tok
        pltpu.sync_copy(oi, tok_hbm.at[pl.ds(ss * L, L)])
        of[...] = new_sco
        pltpu.sync_copy(of, sco_hbm.at[pl.ds(ss * L, L)])
        of[...] = top_vals
        pltpu.sync_copy(of, ssc_hbm.at[pl.ds(ss * L, L)])
        oi[...] = new_ord
        pltpu.sync_copy(oi, ord_hbm.at[pl.ds(ss * L, L)])


def kernel(logits, prev_scores, out_sent_scores, fin_pos, step_nb):
    lg = logits[:, -1].astype(jnp.float32).reshape(-1)
    rv, ri = _stage1(lg)
    norm = jnp.full((L,), (jnp.asarray(step_nb, jnp.float32) + 1.0), jnp.float32)
    fin_f = fin_pos.astype(jnp.float32)
    tokp, scop, sscp, ordp = _stage2(
        rv, ri,
        prev_scores.astype(jnp.float32),
        out_sent_scores.astype(jnp.float32),
        fin_f, norm,
    )
    new_tokens = tokp.reshape(BSZ, L)[:, :BEAM].reshape(-1)
    new_scores = scop.reshape(BSZ, L)[:, :BEAM].reshape(-1)
    new_sent_scores = sscp.reshape(BSZ, L)[:, :BEAM].reshape(-1)
    new_order = ordp.reshape(BSZ, L)[:, :BEAM].reshape(-1)
    return new_tokens, new_scores, new_sent_scores, new_order


# speedup vs baseline: 1.0268x; 1.0268x over previous
import functools

import jax
import jax.numpy as jnp
from jax import lax
from jax.experimental import pallas as pl
from jax.experimental.pallas import tpu as pltpu
from jax.experimental.pallas import tpu_sc as plsc

BEAM = 4
PAD_IDX = 1
EOS_IDX = 2
VOCAB = 1_000_000
BSZ = 16
NROWS = BSZ * BEAM
NC, NS, L = 2, 16, 16
NW = NC * NS
ROWS_PER_W = NROWS // NW
CHUNK = 20_000
NCHUNK = VOCAB // CHUNK
STEPS = CHUNK // L
UNR = 10
NEG = float("-inf")
NEG_I32 = -2147483647

_mesh = plsc.VectorSubcoreMesh(
    core_axis_name="c", subcore_axis_name="s", num_cores=NC, num_subcores=NS
)


def _iota():
    return lax.iota(jnp.int32, L)


def _splat_f(x):
    return jnp.full((L,), x, jnp.float32)


def _extract_f(vec, lane):
    return jnp.max(jnp.where(_iota() == lane, vec, NEG))


@functools.partial(
    pl.kernel,
    out_type=[
        jax.ShapeDtypeStruct((NROWS * L,), jnp.float32),
        jax.ShapeDtypeStruct((NROWS * L,), jnp.int32),
    ],
    mesh=_mesh,
    compiler_params=pltpu.CompilerParams(needs_layout_passes=False),
    scratch_types=[
        pltpu.VMEM((CHUNK,), jnp.float32),
        pltpu.VMEM((CHUNK,), jnp.float32),
        pltpu.VMEM((L,), jnp.float32),
        pltpu.VMEM((L,), jnp.int32),
        pltpu.VMEM((L,), jnp.float32),
        pltpu.VMEM((L,), jnp.int32),
        pltpu.VMEM((L,), jnp.float32),
        pltpu.VMEM((L,), jnp.int32),
        pltpu.SemaphoreType.DMA,
    ],
)
def _stage1(lg_hbm, rv_hbm, ri_hbm, bufa, bufb, rv, ri, gf, gi, ovf, ovi, sem):
    wid = lax.axis_index("s") * NC + lax.axis_index("c")
    io = _iota()

    def drill(bufref, c):
        def body(carry):
            def sa(i, mcar):
                mv, mi = mcar
                base = i * (UNR * L)
                for u in range(UNR):
                    v = bufref[pl.ds(base + u * L, L)]
                    take = v > mv
                    mv = jnp.where(take, v, mv)
                    mi = jnp.where(take, jnp.full((L,), i * UNR + u, jnp.int32), mi)
                return mv, mi

            mv, mi = lax.fori_loop(
                0, STEPS // UNR, sa, (_splat_f(NEG), jnp.zeros((L,), jnp.int32))
            )
            beat = jnp.max(mv) > _extract_f(rv[...], jnp.int32(3))

            @pl.when(beat)
            def _():
                sv, sl = plsc.sort_key_val(mv, io, descending=True)
                gi[...] = mi
                steps = plsc.load_gather(gi, [sl])
                vidx = c * CHUNK + steps * L + sl
                lo = jnp.bitwise_and(io, 3)
                gf[...] = sv
                cva = plsc.load_gather(gf, [lo])
                gi[...] = vidx
                cia = plsc.load_gather(gi, [lo])
                keys = jnp.where(io < 4, rv[...], jnp.where(io < 8, cva, NEG))
                vals = jnp.where(io < 4, ri[...], jnp.where(io < 8, cia, 0))
                sk, skv = plsc.sort_key_val(keys, vals, descending=True)
                rv[...] = jnp.where(io < 4, sk, NEG)
                ri[...] = skv
                plsc.store_scatter(
                    bufref, [steps * L + sl], _splat_f(NEG), mask=io < 4
                )

            return (beat,)

        lax.while_loop(lambda carry: carry[0], body, (True,))

    def process(bufref, c, e0, e1):
        @pl.when(c == 0)
        def _():
            v0 = bufref[pl.ds(0, L)]
            bufref[pl.ds(0, L)] = jnp.where(io == PAD_IDX, NEG, v0)

        def p(i, carry):
            a0, a1, f0, f1 = carry
            base = i * (UNR * L)
            for u in range(UNR // 2):
                v0 = bufref[pl.ds(base + 2 * u * L, L)]
                v1 = bufref[pl.ds(base + (2 * u + 1) * L, L)]
                a0 = jnp.maximum(a0, v0)
                a1 = jnp.maximum(a1, v1)
                f0 = f0 + jnp.exp(v0)
                f1 = f1 + jnp.exp(v1)
            return a0, a1, f0, f1

        a0, a1, e0, e1 = lax.fori_loop(
            0, STEPS // UNR, p, (_splat_f(NEG), _splat_f(NEG), e0, e1)
        )

        @pl.when(jnp.max(jnp.maximum(a0, a1)) > _extract_f(rv[...], jnp.int32(3)))
        def _():
            drill(bufref, c)

        return e0, e1

    for rr in range(ROWS_PER_W):
        row = wid * ROWS_PER_W + rr
        base = row * VOCAB
        rv[...] = _splat_f(NEG)
        ri[...] = jnp.zeros((L,), jnp.int32)
        pltpu.async_copy(lg_hbm.at[pl.ds(base, CHUNK)], bufa, sem)

        def pair(pp, carry):
            e0, e1 = carry
            for b in range(2):
                bcur, bnxt = (bufa, bufb) if b == 0 else (bufb, bufa)
                c = 2 * pp + b
                pltpu.make_async_copy(lg_hbm.at[pl.ds(0, CHUNK)], bcur, sem).wait()

                @pl.when(c + 1 < NCHUNK)
                def _():
                    pltpu.async_copy(
                        lg_hbm.at[pl.ds(base + (c + 1) * CHUNK, CHUNK)], bnxt, sem
                    )

                e0, e1 = process(bcur, c, e0, e1)
            return e0, e1

        e0, e1 = lax.fori_loop(0, NCHUNK // 2, pair, (_splat_f(0.0), _splat_f(0.0)))

        s_row = jnp.sum(e0 + e1)
        ovf[...] = jnp.where(io == 5, s_row, jnp.where(io == 4, 0.0, rv[...]))
        ovi[...] = ri[...]
        pltpu.sync_copy(ovf, rv_hbm.at[pl.ds(row * L, L)])
        pltpu.sync_copy(ovi, ri_hbm.at[pl.ds(row * L, L)])


@functools.partial(
    pl.kernel,
    out_type=[
        jax.ShapeDtypeStruct((BSZ * L,), jnp.int32),
        jax.ShapeDtypeStruct((BSZ * L,), jnp.float32),
        jax.ShapeDtypeStruct((BSZ * L,), jnp.float32),
        jax.ShapeDtypeStruct((BSZ * L,), jnp.int32),
    ],
    mesh=_mesh,
    compiler_params=pltpu.CompilerParams(needs_layout_passes=False),
    scratch_types=[
        pltpu.VMEM((BEAM * L,), jnp.float32),
        pltpu.VMEM((BEAM * L,), jnp.int32),
        pltpu.VMEM((NROWS,), jnp.float32),
        pltpu.VMEM((NROWS,), jnp.float32),
        pltpu.VMEM((NROWS,), jnp.float32),
        pltpu.VMEM((L,), jnp.float32),
        pltpu.VMEM((L,), jnp.float32),
        pltpu.VMEM((L,), jnp.int32),
        pltpu.VMEM((L,), jnp.float32),
        pltpu.VMEM((L,), jnp.int32),
    ],
)
def _stage2(
    rv_hbm, ri_hbm, prev_hbm, osc_hbm, fin_hbm, norm_hbm,
    tok_hbm, sco_hbm, ssc_hbm, ord_hbm,
    vbuf, ibuf, pbuf, obuf, fbuf, nbuf, gf, gi, of, oi,
):
    wid = lax.axis_index("s") * NC + lax.axis_index("c")

    @pl.when(wid < BSZ)
    def _():
        ss = wid
        pltpu.sync_copy(rv_hbm.at[pl.ds(ss * BEAM * L, BEAM * L)], vbuf)
        pltpu.sync_copy(ri_hbm.at[pl.ds(ss * BEAM * L, BEAM * L)], ibuf)
        pltpu.sync_copy(prev_hbm, pbuf)
        pltpu.sync_copy(osc_hbm, obuf)
        pltpu.sync_copy(fin_hbm, fbuf)
        pltpu.sync_copy(norm_hbm, nbuf)

        io = _iota()
        rloc = jnp.right_shift(io, 2)
        slot = jnp.bitwise_and(io, 3)
        rbase = rloc * L
        cv = plsc.load_gather(vbuf, [rbase + slot])
        tok = plsc.load_gather(ibuf, [rbase + slot])
        m = plsc.load_gather(vbuf, [rbase + 4])
        s = plsc.load_gather(vbuf, [rbase + 5])

        bits = plsc.bitcast(s, jnp.int32)
        ef = (jnp.right_shift(bits, 23) - 127).astype(jnp.float32)
        mant = plsc.bitcast(
            jnp.bitwise_or(
                jnp.bitwise_and(bits, jnp.int32(0x007FFFFF)), jnp.int32(0x3F800000)
            ),
            jnp.float32,
        )
        t = mant - 1.0
        x = ef * jnp.float32(0.6931471805599453) + t * (1.0 - t * (0.5 - t / 3.0))
        for _ in range(3):
            x = x + s * jnp.exp(-x) - 1.0
        lse = m + x

        grow = ss * BEAM + rloc
        prev = plsc.load_gather(pbuf, [grow])
        osc = plsc.load_gather(obuf, [grow])
        fin = plsc.load_gather(fbuf, [grow]) > 0.0
        norm = nbuf[...]

        cs = cv - lse + prev
        css = cs / norm
        tok = jnp.where(fin, jnp.int32(EOS_IDX), tok)
        cs = jnp.where(fin, NEG, cs)
        first = slot == 0
        css = jnp.where(fin, jnp.where(first, osc, NEG), css)

        top_vals, top_cands = plsc.sort_key_val(css, io, descending=True)

        gi[...] = tok
        new_tok = plsc.load_gather(gi, [top_cands])
        gf[...] = cs
        new_sco = plsc.load_gather(gf, [top_cands])
        new_ord = jnp.right_shift(top_cands, 2) + ss * BEAM

        oi[...] = new_tok
        pltpu.sync_copy(oi, tok_hbm.at[pl.ds(ss * L, L)])
        of[...] = new_sco
        pltpu.sync_copy(of, sco_hbm.at[pl.ds(ss * L, L)])
        of[...] = top_vals
        pltpu.sync_copy(of, ssc_hbm.at[pl.ds(ss * L, L)])
        oi[...] = new_ord
        pltpu.sync_copy(oi, ord_hbm.at[pl.ds(ss * L, L)])


def kernel(logits, prev_scores, out_sent_scores, fin_pos, step_nb):
    lg = logits[:, -1].astype(jnp.float32).reshape(-1)
    rv, ri = _stage1(lg)
    norm = jnp.full((L,), (jnp.asarray(step_nb, jnp.float32) + 1.0), jnp.float32)
    fin_f = fin_pos.astype(jnp.float32)
    tokp, scop, sscp, ordp = _stage2(
        rv, ri,
        prev_scores.astype(jnp.float32),
        out_sent_scores.astype(jnp.float32),
        fin_f, norm,
    )
    new_tokens = tokp.reshape(BSZ, L)[:, :BEAM].reshape(-1)
    new_scores = scop.reshape(BSZ, L)[:, :BEAM].reshape(-1)
    new_sent_scores = sscp.reshape(BSZ, L)[:, :BEAM].reshape(-1)
    new_order = ordp.reshape(BSZ, L)[:, :BEAM].reshape(-1)
    return new_tokens, new_scores, new_sent_scores, new_order


# speedup vs baseline: 11.8797x; 11.5692x over previous
import functools

import jax
import jax.numpy as jnp
from jax import lax
from jax.experimental import pallas as pl
from jax.experimental.pallas import tpu as pltpu
from jax.experimental.pallas import tpu_sc as plsc

BEAM = 4
PAD_IDX = 1
EOS_IDX = 2
VOCAB = 1_000_000
BSZ = 16
NROWS = BSZ * BEAM
NC, NS, L = 2, 16, 16
NW = NC * NS
NG = 8
NQ = 4
QW = 249_984
CW = 3_968
NCH = QW // CW
TAIL0 = NQ * QW
TW = VOCAB - TAIL0
VPR = CW // L
UNR = 8
ITER = VPR // UNR
NEG = float("-inf")

_mesh = plsc.VectorSubcoreMesh(
    core_axis_name="c", subcore_axis_name="s", num_cores=NC, num_subcores=NS
)


def _iota():
    return lax.iota(jnp.int32, L)


def _splat_f(x):
    return jnp.full((L,), x, jnp.float32)


def _extract_f(vec, lane):
    return jnp.max(jnp.where(_iota() == lane, vec, NEG))


@functools.partial(
    pl.kernel,
    out_type=[
        jax.ShapeDtypeStruct((NROWS * NQ * L,), jnp.float32),
        jax.ShapeDtypeStruct((NROWS * NQ * L,), jnp.int32),
    ],
    mesh=_mesh,
    compiler_params=pltpu.CompilerParams(needs_layout_passes=False),
    scratch_types=[
        pltpu.VMEM((8, CW), jnp.float32),
        pltpu.VMEM((8, CW), jnp.float32),
        pltpu.VMEM((8, TW), jnp.float32),
        pltpu.VMEM((8 * L,), jnp.float32),
        pltpu.VMEM((8 * L,), jnp.int32),
        pltpu.VMEM((8 * L,), jnp.float32),
        pltpu.VMEM((L,), jnp.float32),
        pltpu.VMEM((L,), jnp.int32),
        pltpu.VMEM((L,), jnp.float32),
        pltpu.VMEM((L,), jnp.int32),
        pltpu.SemaphoreType.DMA,
    ],
)
def _stage1(lg_hbm, rv_hbm, ri_hbm, bufa, bufb, tbuf, rv8, ri8, es8, gf, gi, ovf, ovi, sem):
    wid = lax.axis_index("s") * NC + lax.axis_index("c")
    io = _iota()
    g = jnp.right_shift(wid, 2)
    q = jnp.bitwise_and(wid, 3)
    qbase = q * QW
    row0 = pl.multiple_of(g * 8, 8)

    for r in range(8):
        rv8[pl.ds(r * L, L)] = _splat_f(NEG)
        ri8[pl.ds(r * L, L)] = jnp.zeros((L,), jnp.int32)
        es8[pl.ds(r * L, L)] = _splat_f(0.0)

    def drill(bufref, r, iters, unr, colbase):
        rslice = pl.ds(r * L, L)

        def body(carry):
            def sa(i, mcar):
                mv, mi = mcar
                base = i * (unr * L)
                for u in range(unr):
                    v = bufref[r, pl.ds(base + u * L, L)]
                    take = v > mv
                    mv = jnp.where(take, v, mv)
                    mi = jnp.where(take, jnp.full((L,), i * unr + u, jnp.int32), mi)
                return mv, mi

            mv, mi = lax.fori_loop(
                0, iters, sa, (_splat_f(NEG), jnp.zeros((L,), jnp.int32))
            )
            beat = jnp.max(mv) > _extract_f(rv8[rslice], jnp.int32(3))

            @pl.when(beat)
            def _():
                sv, sl = plsc.sort_key_val(mv, io, descending=True)
                gi[...] = mi
                steps = plsc.load_gather(gi, [sl])
                vidx = colbase + steps * L + sl
                lo = jnp.bitwise_and(io, 3)
                gf[...] = sv
                cva = plsc.load_gather(gf, [lo])
                gi[...] = vidx
                cia = plsc.load_gather(gi, [lo])
                keys = jnp.where(io < 4, rv8[rslice], jnp.where(io < 8, cva, NEG))
                vals = jnp.where(io < 4, ri8[rslice], jnp.where(io < 8, cia, 0))
                sk, skv = plsc.sort_key_val(keys, vals, descending=True)
                rv8[rslice] = jnp.where(io < 4, sk, NEG)
                ri8[rslice] = skv
                plsc.store_scatter(
                    bufref,
                    [jnp.full((L,), r, jnp.int32), steps * L + sl],
                    _splat_f(NEG),
                    mask=io < 4,
                )

            return (beat,)

        lax.while_loop(lambda carry: carry[0], body, (True,))

    def process(bufref, ch):
        @pl.when(jnp.logical_and(ch == 0, q == 0))
        def _():
            for r in range(8):
                v0 = bufref[r, pl.ds(0, L)]
                bufref[r, pl.ds(0, L)] = jnp.where(io == PAD_IDX, NEG, v0)

        for r in range(8):
            rslice = pl.ds(r * L, L)

            def p(i, carry):
                a0, a1, f0, f1 = carry
                base = i * (UNR * L)
                for u in range(UNR // 2):
                    v0 = bufref[r, pl.ds(base + 2 * u * L, L)]
                    v1 = bufref[r, pl.ds(base + (2 * u + 1) * L, L)]
                    a0 = jnp.maximum(a0, v0)
                    a1 = jnp.maximum(a1, v1)
                    f0 = f0 + jnp.exp(v0)
                    f1 = f1 + jnp.exp(v1)
                return a0, a1, f0, f1

            a0, a1, f0, f1 = lax.fori_loop(
                0, ITER, p, (_splat_f(NEG), _splat_f(NEG), es8[rslice], _splat_f(0.0))
            )
            es8[rslice] = f0 + f1

            @pl.when(jnp.max(jnp.maximum(a0, a1)) > _extract_f(rv8[rslice], jnp.int32(3)))
            def _():
                drill(bufref, r, ITER, UNR, qbase + ch * CW)

    def src(ch):
        return lg_hbm.at[pl.ds(row0, 8), pl.ds(qbase + ch * CW, CW)]

    tail_src = lg_hbm.at[pl.ds(row0, 8), pl.ds(TAIL0, TW)]

    pltpu.async_copy(src(0), bufa, sem)

    def pairbody(pp, carry):
        for b in range(2):
            bcur, bnxt = (bufa, bufb) if b == 0 else (bufb, bufa)
            ch = 2 * pp + b
            pltpu.make_async_copy(src(ch), bcur, sem).wait()

            @pl.when(ch + 1 < NCH)
            def _():
                pltpu.async_copy(src(ch + 1), bnxt, sem)

            process(bcur, ch)
        return carry

    lax.fori_loop(0, (NCH - 1) // 2, pairbody, 0)

    pltpu.make_async_copy(src(NCH - 1), bufa, sem).wait()

    @pl.when(q == 3)
    def _():
        pltpu.async_copy(tail_src, tbuf, sem)

    process(bufa, NCH - 1)

    @pl.when(q == 3)
    def _():
        pltpu.make_async_copy(tail_src, tbuf, sem).wait()
        for r in range(8):
            rslice = pl.ds(r * L, L)
            a = _splat_f(NEG)
            e = es8[rslice]
            for v in range(TW // L):
                vv = tbuf[r, pl.ds(v * L, L)]
                a = jnp.maximum(a, vv)
                e = e + jnp.exp(vv)
            es8[rslice] = e

            @pl.when(jnp.max(a) > _extract_f(rv8[rslice], jnp.int32(3)))
            def _():
                drill(tbuf, r, 1, TW // L, TAIL0)

    for r in range(8):
        rslice = pl.ds(r * L, L)
        s_row = jnp.sum(es8[rslice])
        ovf[...] = jnp.where(io == 5, s_row, jnp.where(io == 4, 0.0, rv8[rslice]))
        ovi[...] = ri8[rslice]
        slot = ((g * 8 + r) * NQ + q) * L
        pltpu.sync_copy(ovf, rv_hbm.at[pl.ds(slot, L)])
        pltpu.sync_copy(ovi, ri_hbm.at[pl.ds(slot, L)])


@functools.partial(
    pl.kernel,
    out_type=[
        jax.ShapeDtypeStruct((BSZ * L,), jnp.int32),
        jax.ShapeDtypeStruct((BSZ * L,), jnp.float32),
        jax.ShapeDtypeStruct((BSZ * L,), jnp.float32),
        jax.ShapeDtypeStruct((BSZ * L,), jnp.int32),
    ],
    mesh=_mesh,
    compiler_params=pltpu.CompilerParams(needs_layout_passes=False),
    scratch_types=[
        pltpu.VMEM((4 * NQ * L,), jnp.float32),
        pltpu.VMEM((4 * NQ * L,), jnp.int32),
        pltpu.VMEM((4 * L,), jnp.float32),
        pltpu.VMEM((4 * L,), jnp.int32),
        pltpu.VMEM((NROWS,), jnp.float32),
        pltpu.VMEM((NROWS,), jnp.float32),
        pltpu.VMEM((NROWS,), jnp.float32),
        pltpu.VMEM((L,), jnp.float32),
        pltpu.VMEM((L,), jnp.float32),
        pltpu.VMEM((L,), jnp.int32),
        pltpu.VMEM((L,), jnp.float32),
        pltpu.VMEM((L,), jnp.int32),
    ],
)
def _stage2(
    rv_hbm, ri_hbm, prev_hbm, osc_hbm, fin_hbm, norm_hbm,
    tok_hbm, sco_hbm, ssc_hbm, ord_hbm,
    vbuf, ibuf, c4f, c4i, pbuf, obuf, fbuf, nbuf, gf, gi, of, oi,
):
    wid = lax.axis_index("s") * NC + lax.axis_index("c")

    @pl.when(wid < BSZ)
    def _():
        ss = wid
        pltpu.sync_copy(rv_hbm.at[pl.ds(ss * 4 * NQ * L, 4 * NQ * L)], vbuf)
        pltpu.sync_copy(ri_hbm.at[pl.ds(ss * 4 * NQ * L, 4 * NQ * L)], ibuf)
        pltpu.sync_copy(prev_hbm, pbuf)
        pltpu.sync_copy(osc_hbm, obuf)
        pltpu.sync_copy(fin_hbm, fbuf)
        pltpu.sync_copy(norm_hbm, nbuf)

        io = _iota()
        hi = jnp.right_shift(io, 2)
        lo = jnp.bitwise_and(io, 3)

        for r in range(4):
            idxv = r * (NQ * L) + hi * L + lo
            vals16 = plsc.load_gather(vbuf, [idxv])
            ivals16 = plsc.load_gather(ibuf, [idxv])
            sv, si = plsc.sort_key_val(vals16, ivals16, descending=True)
            sidx = r * (NQ * L) + lo * L + 5
            sg = plsc.load_gather(vbuf, [sidx])
            s_r = jnp.sum(jnp.where(io < 4, sg, 0.0))
            c4f[pl.ds(r * L, L)] = jnp.where(
                io < 4, sv, jnp.where(io == 5, s_r, 0.0)
            )
            c4i[pl.ds(r * L, L)] = si

        rloc = hi
        slot = lo
        rbase = rloc * L
        cv = plsc.load_gather(c4f, [rbase + slot])
        tok = plsc.load_gather(c4i, [rbase + slot])
        s = plsc.load_gather(c4f, [rbase + 5])

        bits = plsc.bitcast(s, jnp.int32)
        ef = (jnp.right_shift(bits, 23) - 127).astype(jnp.float32)
        mant = plsc.bitcast(
            jnp.bitwise_or(
                jnp.bitwise_and(bits, jnp.int32(0x007FFFFF)), jnp.int32(0x3F800000)
            ),
            jnp.float32,
        )
        t = mant - 1.0
        x = ef * jnp.float32(0.6931471805599453) + t * (1.0 - t * (0.5 - t / 3.0))
        for _ in range(3):
            x = x + s * jnp.exp(-x) - 1.0
        lse = x

        grow = ss * BEAM + rloc
        prev = plsc.load_gather(pbuf, [grow])
        osc = plsc.load_gather(obuf, [grow])
        fin = plsc.load_gather(fbuf, [grow]) > 0.0
        norm = nbuf[...]

        cs = cv - lse + prev
        css = cs / norm
        tok = jnp.where(fin, jnp.int32(EOS_IDX), tok)
        cs = jnp.where(fin, NEG, cs)
        first = slot == 0
        css = jnp.where(fin, jnp.where(first, osc, NEG), css)

        top_vals, top_cands = plsc.sort_key_val(css, io, descending=True)

        gi[...] = tok
        new_tok = plsc.load_gather(gi, [top_cands])
        gf[...] = cs
        new_sco = plsc.load_gather(gf, [top_cands])
        new_ord = jnp.right_shift(top_cands, 2) + ss * BEAM

        oi[...] = new_tok
        pltpu.sync_copy(oi, tok_hbm.at[pl.ds(ss * L, L)])
        of[...] = new_sco
        pltpu.sync_copy(of, sco_hbm.at[pl.ds(ss * L, L)])
        of[...] = top_vals
        pltpu.sync_copy(of, ssc_hbm.at[pl.ds(ss * L, L)])
        oi[...] = new_ord
        pltpu.sync_copy(oi, ord_hbm.at[pl.ds(ss * L, L)])


def kernel(logits, prev_scores, out_sent_scores, fin_pos, step_nb):
    lg = logits[:, -1].astype(jnp.float32)
    rv, ri = _stage1(lg)
    norm = jnp.full((L,), (jnp.asarray(step_nb, jnp.float32) + 1.0), jnp.float32)
    fin_f = fin_pos.astype(jnp.float32)
    tokp, scop, sscp, ordp = _stage2(
        rv, ri,
        prev_scores.astype(jnp.float32),
        out_sent_scores.astype(jnp.float32),
        fin_f, norm,
    )
    new_tokens = tokp.reshape(BSZ, L)[:, :BEAM].reshape(-1)
    new_scores = scop.reshape(BSZ, L)[:, :BEAM].reshape(-1)
    new_sent_scores = sscp.reshape(BSZ, L)[:, :BEAM].reshape(-1)
    new_order = ordp.reshape(BSZ, L)[:, :BEAM].reshape(-1)
    return new_tokens, new_scores, new_sent_scores, new_order


# speedup vs baseline: 20.0641x; 1.6889x over previous
import functools

import jax
import jax.numpy as jnp
from jax import lax
from jax.experimental import pallas as pl
from jax.experimental.pallas import tpu as pltpu
from jax.experimental.pallas import tpu_sc as plsc

BEAM = 4
PAD_IDX = 1
EOS_IDX = 2
VOCAB = 1_000_000
BSZ = 16
NROWS = BSZ * BEAM
NC, NS, L = 2, 16, 16
NW = NC * NS
NG = 8
NQ = 4
QW = 249_984
CW = 3_968
NCH = QW // CW
TAIL0 = NQ * QW
TW = VOCAB - TAIL0
VPR = CW // L
UNR = 8
ITER = VPR // UNR
NEG = float("-inf")

_mesh = plsc.VectorSubcoreMesh(
    core_axis_name="c", subcore_axis_name="s", num_cores=NC, num_subcores=NS
)


def _iota():
    return lax.iota(jnp.int32, L)


def _splat_f(x):
    return jnp.full((L,), x, jnp.float32)


def _extract_f(vec, lane):
    return jnp.max(jnp.where(_iota() == lane, vec, NEG))


@functools.partial(
    pl.kernel,
    out_type=[
        jax.ShapeDtypeStruct((NROWS * NQ * L,), jnp.float32),
        jax.ShapeDtypeStruct((NROWS * NQ * L,), jnp.int32),
    ],
    mesh=_mesh,
    compiler_params=pltpu.CompilerParams(needs_layout_passes=False),
    scratch_types=[
        pltpu.VMEM((8, CW), jnp.float32),
        pltpu.VMEM((8, CW), jnp.float32),
        pltpu.VMEM((8, TW), jnp.float32),
        pltpu.VMEM((8 * L,), jnp.float32),
        pltpu.VMEM((8 * L,), jnp.int32),
        pltpu.VMEM((8 * L,), jnp.float32),
        pltpu.VMEM((L,), jnp.float32),
        pltpu.VMEM((L,), jnp.int32),
        pltpu.VMEM((L,), jnp.float32),
        pltpu.VMEM((L,), jnp.int32),
        pltpu.SemaphoreType.DMA,
    ],
)
def _stage1(lg_hbm, rv_hbm, ri_hbm, bufa, bufb, tbuf, rv8, ri8, es8, gf, gi, ovf, ovi, sem):
    wid = lax.axis_index("s") * NC + lax.axis_index("c")
    io = _iota()
    g = jnp.right_shift(wid, 2)
    q = jnp.bitwise_and(wid, 3)
    qbase = q * QW
    row0 = pl.multiple_of(g * 8, 8)

    for r in range(8):
        rv8[pl.ds(r * L, L)] = _splat_f(NEG)
        ri8[pl.ds(r * L, L)] = jnp.zeros((L,), jnp.int32)
        es8[pl.ds(r * L, L)] = _splat_f(0.0)

    def drill(bufref, r, iters, unr, colbase):
        rslice = pl.ds(r * L, L)

        def body(carry):
            def sa(i, mcar):
                mv, mi = mcar
                base = i * (unr * L)
                for u in range(unr):
                    v = bufref[r, pl.ds(base + u * L, L)]
                    take = v > mv
                    mv = jnp.where(take, v, mv)
                    mi = jnp.where(take, jnp.full((L,), i * unr + u, jnp.int32), mi)
                return mv, mi

            mv, mi = lax.fori_loop(
                0, iters, sa, (_splat_f(NEG), jnp.zeros((L,), jnp.int32))
            )
            beat = jnp.max(mv) > _extract_f(rv8[rslice], jnp.int32(3))

            @pl.when(beat)
            def _():
                sv, sl = plsc.sort_key_val(mv, io, descending=True)
                gi[...] = mi
                steps = plsc.load_gather(gi, [sl])
                vidx = colbase + steps * L + sl
                lo = jnp.bitwise_and(io, 3)
                gf[...] = sv
                cva = plsc.load_gather(gf, [lo])
                gi[...] = vidx
                cia = plsc.load_gather(gi, [lo])
                keys = jnp.where(io < 4, rv8[rslice], jnp.where(io < 8, cva, NEG))
                vals = jnp.where(io < 4, ri8[rslice], jnp.where(io < 8, cia, 0))
                sk, skv = plsc.sort_key_val(keys, vals, descending=True)
                rv8[rslice] = jnp.where(io < 4, sk, NEG)
                ri8[rslice] = skv
                plsc.store_scatter(
                    bufref,
                    [jnp.full((L,), r, jnp.int32), steps * L + sl],
                    _splat_f(NEG),
                    mask=io < 4,
                )

            return (beat,)

        lax.while_loop(lambda carry: carry[0], body, (True,))

    def process(bufref, ch):
        @pl.when(jnp.logical_and(ch == 0, q == 0))
        def _():
            for r in range(8):
                v0 = bufref[r, pl.ds(0, L)]
                bufref[r, pl.ds(0, L)] = jnp.where(io == PAD_IDX, NEG, v0)

        for r in range(8):
            rslice = pl.ds(r * L, L)

            def p(i, carry):
                a0, a1, f0, f1 = carry
                base = i * (UNR * L)
                for u in range(UNR // 2):
                    v0 = bufref[r, pl.ds(base + 2 * u * L, L)]
                    v1 = bufref[r, pl.ds(base + (2 * u + 1) * L, L)]
                    a0 = jnp.maximum(a0, v0)
                    a1 = jnp.maximum(a1, v1)
                    f0 = f0 + jnp.exp(v0)
                    f1 = f1 + jnp.exp(v1)
                return a0, a1, f0, f1

            a0, a1, f0, f1 = lax.fori_loop(
                0, ITER, p, (_splat_f(NEG), _splat_f(NEG), es8[rslice], _splat_f(0.0))
            )
            es8[rslice] = f0 + f1

            @pl.when(jnp.max(jnp.maximum(a0, a1)) > _extract_f(rv8[rslice], jnp.int32(3)))
            def _():
                drill(bufref, r, ITER, UNR, qbase + ch * CW)

    def src(ch):
        return lg_hbm.at[pl.ds(row0, 8), 0, pl.ds(qbase + ch * CW, CW)]

    tail_src = lg_hbm.at[pl.ds(row0, 8), 0, pl.ds(TAIL0, TW)]

    pltpu.async_copy(src(0), bufa, sem)

    def pairbody(pp, carry):
        for b in range(2):
            bcur, bnxt = (bufa, bufb) if b == 0 else (bufb, bufa)
            ch = 2 * pp + b
            pltpu.make_async_copy(src(ch), bcur, sem).wait()

            @pl.when(ch + 1 < NCH)
            def _():
                pltpu.async_copy(src(ch + 1), bnxt, sem)

            process(bcur, ch)
        return carry

    lax.fori_loop(0, (NCH - 1) // 2, pairbody, 0)

    pltpu.make_async_copy(src(NCH - 1), bufa, sem).wait()

    @pl.when(q == 3)
    def _():
        pltpu.async_copy(tail_src, tbuf, sem)

    process(bufa, NCH - 1)

    @pl.when(q == 3)
    def _():
        pltpu.make_async_copy(tail_src, tbuf, sem).wait()
        for r in range(8):
            rslice = pl.ds(r * L, L)
            a = _splat_f(NEG)
            e = es8[rslice]
            for v in range(TW // L):
                vv = tbuf[r, pl.ds(v * L, L)]
                a = jnp.maximum(a, vv)
                e = e + jnp.exp(vv)
            es8[rslice] = e

            @pl.when(jnp.max(a) > _extract_f(rv8[rslice], jnp.int32(3)))
            def _():
                drill(tbuf, r, 1, TW // L, TAIL0)

    for r in range(8):
        rslice = pl.ds(r * L, L)
        s_row = jnp.sum(es8[rslice])
        ovf[...] = jnp.where(io == 5, s_row, jnp.where(io == 4, 0.0, rv8[rslice]))
        ovi[...] = ri8[rslice]
        slot = ((g * 8 + r) * NQ + q) * L
        pltpu.sync_copy(ovf, rv_hbm.at[pl.ds(slot, L)])
        pltpu.sync_copy(ovi, ri_hbm.at[pl.ds(slot, L)])


@functools.partial(
    pl.kernel,
    out_type=[
        jax.ShapeDtypeStruct((BSZ * L,), jnp.int32),
        jax.ShapeDtypeStruct((BSZ * L,), jnp.float32),
        jax.ShapeDtypeStruct((BSZ * L,), jnp.float32),
        jax.ShapeDtypeStruct((BSZ * L,), jnp.int32),
    ],
    mesh=_mesh,
    compiler_params=pltpu.CompilerParams(needs_layout_passes=False),
    scratch_types=[
        pltpu.VMEM((4 * NQ * L,), jnp.float32),
        pltpu.VMEM((4 * NQ * L,), jnp.int32),
        pltpu.VMEM((4 * L,), jnp.float32),
        pltpu.VMEM((4 * L,), jnp.int32),
        pltpu.VMEM((NROWS,), jnp.float32),
        pltpu.VMEM((NROWS,), jnp.float32),
        pltpu.VMEM((NROWS,), jnp.float32),
        pltpu.VMEM((L,), jnp.float32),
        pltpu.VMEM((L,), jnp.float32),
        pltpu.VMEM((L,), jnp.int32),
        pltpu.VMEM((L,), jnp.float32),
        pltpu.VMEM((L,), jnp.int32),
    ],
)
def _stage2(
    rv_hbm, ri_hbm, prev_hbm, osc_hbm, fin_hbm, norm_hbm,
    tok_hbm, sco_hbm, ssc_hbm, ord_hbm,
    vbuf, ibuf, c4f, c4i, pbuf, obuf, fbuf, nbuf, gf, gi, of, oi,
):
    wid = lax.axis_index("s") * NC + lax.axis_index("c")

    @pl.when(wid < BSZ)
    def _():
        ss = wid
        pltpu.sync_copy(rv_hbm.at[pl.ds(ss * 4 * NQ * L, 4 * NQ * L)], vbuf)
        pltpu.sync_copy(ri_hbm.at[pl.ds(ss * 4 * NQ * L, 4 * NQ * L)], ibuf)
        pltpu.sync_copy(prev_hbm, pbuf)
        pltpu.sync_copy(osc_hbm, obuf)
        pltpu.sync_copy(fin_hbm, fbuf)
        pltpu.sync_copy(norm_hbm, nbuf)

        io = _iota()
        hi = jnp.right_shift(io, 2)
        lo = jnp.bitwise_and(io, 3)

        for r in range(4):
            idxv = r * (NQ * L) + hi * L + lo
            vals16 = plsc.load_gather(vbuf, [idxv])
            ivals16 = plsc.load_gather(ibuf, [idxv])
            sv, si = plsc.sort_key_val(vals16, ivals16, descending=True)
            sidx = r * (NQ * L) + lo * L + 5
            sg = plsc.load_gather(vbuf, [sidx])
            s_r = jnp.sum(jnp.where(io < 4, sg, 0.0))
            c4f[pl.ds(r * L, L)] = jnp.where(
                io < 4, sv, jnp.where(io == 5, s_r, 0.0)
            )
            c4i[pl.ds(r * L, L)] = si

        rloc = hi
        slot = lo
        rbase = rloc * L
        cv = plsc.load_gather(c4f, [rbase + slot])
        tok = plsc.load_gather(c4i, [rbase + slot])
        s = plsc.load_gather(c4f, [rbase + 5])

        bits = plsc.bitcast(s, jnp.int32)
        ef = (jnp.right_shift(bits, 23) - 127).astype(jnp.float32)
        mant = plsc.bitcast(
            jnp.bitwise_or(
                jnp.bitwise_and(bits, jnp.int32(0x007FFFFF)), jnp.int32(0x3F800000)
            ),
            jnp.float32,
        )
        t = mant - 1.0
        x = ef * jnp.float32(0.6931471805599453) + t * (1.0 - t * (0.5 - t / 3.0))
        for _ in range(3):
            x = x + s * jnp.exp(-x) - 1.0
        lse = x

        grow = ss * BEAM + rloc
        prev = plsc.load_gather(pbuf, [grow])
        osc = plsc.load_gather(obuf, [grow])
        fin = plsc.load_gather(fbuf, [grow]) > 0.0
        norm = nbuf[...]

        cs = cv - lse + prev
        css = cs / norm
        tok = jnp.where(fin, jnp.int32(EOS_IDX), tok)
        cs = jnp.where(fin, NEG, cs)
        first = slot == 0
        css = jnp.where(fin, jnp.where(first, osc, NEG), css)

        top_vals, top_cands = plsc.sort_key_val(css, io, descending=True)

        gi[...] = tok
        new_tok = plsc.load_gather(gi, [top_cands])
        gf[...] = cs
        new_sco = plsc.load_gather(gf, [top_cands])
        new_ord = jnp.right_shift(top_cands, 2) + ss * BEAM

        oi[...] = new_tok
        pltpu.sync_copy(oi, tok_hbm.at[pl.ds(ss * L, L)])
        of[...] = new_sco
        pltpu.sync_copy(of, sco_hbm.at[pl.ds(ss * L, L)])
        of[...] = top_vals
        pltpu.sync_copy(of, ssc_hbm.at[pl.ds(ss * L, L)])
        oi[...] = new_ord
        pltpu.sync_copy(oi, ord_hbm.at[pl.ds(ss * L, L)])


def kernel(logits, prev_scores, out_sent_scores, fin_pos, step_nb):
    lg = logits.astype(jnp.float32)
    rv, ri = _stage1(lg)
    norm = jnp.full((L,), (jnp.asarray(step_nb, jnp.float32) + 1.0), jnp.float32)
    fin_f = fin_pos.astype(jnp.float32)
    tokp, scop, sscp, ordp = _stage2(
        rv, ri,
        prev_scores.astype(jnp.float32),
        out_sent_scores.astype(jnp.float32),
        fin_f, norm,
    )
    new_tokens = tokp.reshape(BSZ, L)[:, :BEAM].reshape(-1)
    new_scores = scop.reshape(BSZ, L)[:, :BEAM].reshape(-1)
    new_sent_scores = sscp.reshape(BSZ, L)[:, :BEAM].reshape(-1)
    new_order = ordp.reshape(BSZ, L)[:, :BEAM].reshape(-1)
    return new_tokens, new_scores, new_sent_scores, new_order
